# baseline (speedup 1.0000x reference)
import functools

import jax
import jax.numpy as jnp
from jax import lax
from jax.experimental import pallas as pl
from jax.experimental.pallas import tpu as pltpu
from jax.experimental.pallas import tpu_sc as plsc

_VOCAB = 1000000
_DIM = 32
_NC, _NS, _L = 2, 16, 16
_NW = _NC * _NS
_BATCH = 16384
_FIELDS = 26
_CHUNK = 512
_TOTAL = _BATCH * _FIELDS
_B_PER_W = _TOTAL // _NW
_CH_PER_W = _B_PER_W // _CHUNK
_CH_PER_F = _BATCH // _CHUNK
_NSLOT = 3

_NA = 15625
_A_PER_W = 489
_NA_PAD = _A_PER_W * _NW
_NB = 65536
_B_COLS_W = _NB // _NW
_SUB = 512


def _compute_indices(v):
    lo = jnp.bitwise_and(v, jnp.int32(0xFFFF))
    hi = jnp.right_shift(v, jnp.int32(16))
    m = hi * jnp.int32(1024)
    q = (m.astype(jnp.float32) * jnp.float32(1.0 / 15625.0)).astype(jnp.int32)
    r = m - q * jnp.int32(15625)
    r = jnp.where(r < 0, r + jnp.int32(15625), r)
    r = jnp.where(r >= jnp.int32(15625), r - jnp.int32(15625), r)
    return r, lo


def _make_stage_kernel():
    mesh = plsc.VectorSubcoreMesh(core_axis_name="c", subcore_axis_name="s")

    @functools.partial(
        pl.kernel,
        out_type=jax.ShapeDtypeStruct((_NB * _DIM,), jnp.float32),
        mesh=mesh,
        compiler_params=pltpu.CompilerParams(use_tc_tiling_on_sc=True,
                                             needs_layout_passes=False),
        scratch_types=[
            pltpu.VMEM((_DIM, _SUB), jnp.float32),
            [pltpu.VMEM((_SUB * _DIM,), jnp.float32)] * 2,
            pltpu.SemaphoreType.DMA,
            [pltpu.SemaphoreType.DMA] * 2,
        ],
    )
    def stage_kernel(tab_t, b_out, bbuf, sbuf, sem_r, sem_w):
        wid = lax.axis_index("s") * _NC + lax.axis_index("c")
        iota32 = lax.iota(jnp.int32, _L) * jnp.int32(_DIM)

        cp_w = [None, None]
        for sub in range(_B_COLS_W // _SUB):
            t = sub % 2
            c0 = wid * _B_COLS_W + sub * _SUB
            cps = []
            for ib in range(_DIM // 8):
                cps.append(pltpu.async_copy(
                    tab_t.at[pl.ds(8 * ib, 8), pl.ds(c0, _SUB)],
                    bbuf.at[pl.ds(8 * ib, 8), :], sem_r))
            for cp in cps:
                cp.wait()
            if cp_w[t] is not None:
                cp_w[t].wait()

            @plsc.parallel_loop(0, _DIM * (_SUB // _L), unroll=8)
            def _tr(i, _t=t):
                d = i % _DIM
                qq = i // _DIM
                vec = bbuf[d, pl.ds(qq * _L, _L)]
                plsc.store_scatter(sbuf[_t],
                                   [iota32 + (qq * _L * _DIM + d)], vec)
            cp_w[t] = pltpu.async_copy(
                sbuf[t], b_out.at[pl.ds(c0 * _DIM, _SUB * _DIM)], sem_w[t])
        for t in range(2):
            if cp_w[t] is not None:
                cp_w[t].wait()

    return stage_kernel


def _make_lookup_kernel():
    mesh = plsc.VectorSubcoreMesh(core_axis_name="c", subcore_axis_name="s")

    @functools.partial(
        pl.kernel,
        out_type=jax.ShapeDtypeStruct(
            (_FIELDS, _DIM // 8, _BATCH // 128, 1024), jnp.float32),
        mesh=mesh,
        compiler_params=pltpu.CompilerParams(use_tc_tiling_on_sc=False,
                                             needs_layout_passes=False),
        scratch_types=[
            pltpu.VMEM((_B_PER_W,), jnp.int32),
            pltpu.VMEM((_CH_PER_W, _CHUNK), jnp.int32),
            pltpu.VMEM((_CH_PER_W, _CHUNK), jnp.int32),
            [pltpu.VMEM((_CHUNK, _DIM), jnp.float32)] * _NSLOT,
            pltpu.VMEM((4 * _DIM * 128,), jnp.float32),
            [pltpu.SemaphoreType.DMA] * _NSLOT,
            pltpu.SemaphoreType.DMA,
        ],
    )
    def lookup_kernel(ids_hbm, a_tab, b_tab, out_hbm,
                      ids_v, idx0_v, idx1_v, rows, tbuf, sem_g, sem_o):
        wid = lax.axis_index("s") * _NC + lax.axis_index("c")
        base = wid * _B_PER_W

        pltpu.sync_copy(ids_hbm.at[pl.ds(base, _B_PER_W)], ids_v)

        @plsc.parallel_loop(0, _B_PER_W // _L, unroll=4)
        def _idx(i):
            c = i // (_CHUNK // _L)
            j = i % (_CHUNK // _L)
            v = ids_v[pl.ds(i * _L, _L)]
            i0, i1 = _compute_indices(v)
            idx0_v[c, pl.ds(j * _L, _L)] = i0
            idx1_v[c, pl.ds(j * _L, _L)] = i1

        iota128 = lax.iota(jnp.int32, _L) * jnp.int32(128)

        cp_a = [None] * _NSLOT
        cp_b = [None] * _NSLOT
        cp_o = [None, None]

        def start_a(c):
            s = c % _NSLOT
            cp_a[s] = pltpu.async_copy(a_tab.at[idx0_v.at[c]], rows[s],
                                       sem_g[s])

        def a_to_b(c):
            s = c % _NSLOT
            cp_a[s].wait()
            cp_b[s] = pltpu.async_copy(b_tab.at[idx1_v.at[c]], rows[s],
                                       sem_g[s], add=True)

        def drain_writes():
            for _ in range(16):
                pltpu.make_async_copy(out_hbm.at[0, 0, 0],
                                      tbuf.at[pl.ds(0, 1024)], sem_o).wait()

        def write_out(c):
            s = c % _NSLOT
            cp_b[s].wait()
            cg = base // _CHUNK + c
            f = cg // _CH_PER_F
            j0 = (cg % _CH_PER_F) * 4
            for half in range(1):

                def blk(q, carry, _s=s, _half=half, _f=f, _j0=j0):
                    jl = _half * 4 + q
                    toff = q * 4096

                    @plsc.parallel_loop(0, 128, unroll=8)
                    def _tr(i):
                        v0 = rows[_s][jl * 128 + i, pl.ds(0, _L)]
                        plsc.store_scatter(tbuf, [iota128 + (toff + i)], v0)
                        v1 = rows[_s][jl * 128 + i, pl.ds(_L, _L)]
                        plsc.store_scatter(
                            tbuf, [iota128 + (toff + _L * 128 + i)], v1)

                    for i in range(_DIM // 8):
                        pltpu.async_copy(
                            tbuf.at[pl.ds(toff + 1024 * i, 1024)],
                            out_hbm.at[_f, i, _j0 + jl],
                            sem_o)
                    return carry

                lax.fori_loop(0, 4, blk, 0)
                drain_writes()
            return

        for c in range(_CH_PER_W + 2):
            if 2 <= c:
                write_out(c - 2)
            if c < _CH_PER_W:
                start_a(c)
            if 1 <= c <= _CH_PER_W:
                a_to_b(c - 1)

    return lookup_kernel


def kernel(ids, table):
    tab_t = jnp.transpose(table)
    ids_flat = jnp.transpose(ids).reshape(_TOTAL)
    a_tab = table[: (_NA - 1) * 64 + 1 : 64]
    b1 = _make_stage_kernel()(tab_t)
    out5 = _make_lookup_kernel()(
        ids_flat,
        a_tab,
        b1.reshape(_NB, _DIM),
    )
    out5 = out5.reshape(_FIELDS, _DIM // 8, _BATCH // 128, 8, 128)
    return out5.transpose(2, 4, 0, 1, 3).reshape(_BATCH, _FIELDS, _DIM)


# speedup vs baseline: 1.0161x; 1.0161x over previous
import functools

import jax
import jax.numpy as jnp
from jax import lax
from jax.experimental import pallas as pl
from jax.experimental.pallas import tpu as pltpu
from jax.experimental.pallas import tpu_sc as plsc

_VOCAB = 1000000
_DIM = 32
_NC, _NS, _L = 2, 16, 16
_NW = _NC * _NS
_BATCH = 16384
_FIELDS = 26
_CHUNK = 1024
_TOTAL = _BATCH * _FIELDS
_B_PER_W = _TOTAL // _NW
_CH_PER_W = _B_PER_W // _CHUNK
_CH_PER_F = _BATCH // _CHUNK
_NSLOT = 2

_NA = 15625
_A_PER_W = 489
_NA_PAD = _A_PER_W * _NW
_NB = 65536
_B_COLS_W = _NB // _NW
_SUB = 512


def _compute_indices(v):
    lo = jnp.bitwise_and(v, jnp.int32(0xFFFF))
    hi = jnp.right_shift(v, jnp.int32(16))
    m = hi * jnp.int32(1024)
    q = (m.astype(jnp.float32) * jnp.float32(1.0 / 15625.0)).astype(jnp.int32)
    r = m - q * jnp.int32(15625)
    r = jnp.where(r < 0, r + jnp.int32(15625), r)
    r = jnp.where(r >= jnp.int32(15625), r - jnp.int32(15625), r)
    return r, lo


def _make_stage_kernel():
    mesh = plsc.VectorSubcoreMesh(core_axis_name="c", subcore_axis_name="s")

    @functools.partial(
        pl.kernel,
        out_type=jax.ShapeDtypeStruct((_NB * _DIM,), jnp.float32),
        mesh=mesh,
        compiler_params=pltpu.CompilerParams(use_tc_tiling_on_sc=True,
                                             needs_layout_passes=False),
        scratch_types=[
            pltpu.VMEM((_DIM, _SUB), jnp.float32),
            [pltpu.VMEM((_SUB * _DIM,), jnp.float32)] * 2,
            pltpu.SemaphoreType.DMA,
            [pltpu.SemaphoreType.DMA] * 2,
        ],
    )
    def stage_kernel(tab_t, b_out, bbuf, sbuf, sem_r, sem_w):
        wid = lax.axis_index("s") * _NC + lax.axis_index("c")
        iota32 = lax.iota(jnp.int32, _L) * jnp.int32(_DIM)

        cp_w = [None, None]
        for sub in range(_B_COLS_W // _SUB):
            t = sub % 2
            c0 = wid * _B_COLS_W + sub * _SUB
            cps = []
            for ib in range(_DIM // 8):
                cps.append(pltpu.async_copy(
                    tab_t.at[pl.ds(8 * ib, 8), pl.ds(c0, _SUB)],
                    bbuf.at[pl.ds(8 * ib, 8), :], sem_r))
            for cp in cps:
                cp.wait()
            if cp_w[t] is not None:
                cp_w[t].wait()

            @plsc.parallel_loop(0, _DIM * (_SUB // _L), unroll=8)
            def _tr(i, _t=t):
                d = i % _DIM
                qq = i // _DIM
                vec = bbuf[d, pl.ds(qq * _L, _L)]
                plsc.store_scatter(sbuf[_t],
                                   [iota32 + (qq * _L * _DIM + d)], vec)
            cp_w[t] = pltpu.async_copy(
                sbuf[t], b_out.at[pl.ds(c0 * _DIM, _SUB * _DIM)], sem_w[t])
        for t in range(2):
            if cp_w[t] is not None:
                cp_w[t].wait()

    return stage_kernel


def _make_lookup_kernel():
    mesh = plsc.VectorSubcoreMesh(core_axis_name="c", subcore_axis_name="s")

    @functools.partial(
        pl.kernel,
        out_type=jax.ShapeDtypeStruct(
            (_FIELDS, _DIM // 8, _BATCH // 128, 1024), jnp.float32),
        mesh=mesh,
        compiler_params=pltpu.CompilerParams(use_tc_tiling_on_sc=False,
                                             needs_layout_passes=False),
        scratch_types=[
            pltpu.VMEM((_B_PER_W,), jnp.int32),
            pltpu.VMEM((_CH_PER_W, _CHUNK), jnp.int32),
            pltpu.VMEM((_CH_PER_W, _CHUNK), jnp.int32),
            [pltpu.VMEM((_CHUNK, _DIM), jnp.float32)] * _NSLOT,
            pltpu.VMEM((4 * _DIM * 128,), jnp.float32),
            [pltpu.SemaphoreType.DMA] * _NSLOT,
            pltpu.SemaphoreType.DMA,
        ],
    )
    def lookup_kernel(ids_hbm, a_tab, b_tab, out_hbm,
                      ids_v, idx0_v, idx1_v, rows, tbuf, sem_g, sem_o):
        wid = lax.axis_index("s") * _NC + lax.axis_index("c")
        base = wid * _B_PER_W

        pltpu.sync_copy(ids_hbm.at[pl.ds(base, _B_PER_W)], ids_v)

        @plsc.parallel_loop(0, _B_PER_W // _L, unroll=8)
        def _idx(i):
            c = i // (_CHUNK // _L)
            j = i % (_CHUNK // _L)
            v = ids_v[pl.ds(i * _L, _L)]
            i0, i1 = _compute_indices(v)
            idx0_v[c, pl.ds(j * _L, _L)] = i0
            idx1_v[c, pl.ds(j * _L, _L)] = i1

        iota128 = lax.iota(jnp.int32, _L) * jnp.int32(128)

        cp_a = [None] * _NSLOT
        cp_b = [None] * _NSLOT
        cp_o = [None, None]

        def start_a(c):
            s = c % _NSLOT
            cp_a[s] = pltpu.async_copy(a_tab.at[idx0_v.at[c]], rows[s],
                                       sem_g[s])

        def a_to_b(c):
            s = c % _NSLOT
            cp_a[s].wait()
            cp_b[s] = pltpu.async_copy(b_tab.at[idx1_v.at[c]], rows[s],
                                       sem_g[s], add=True)

        def drain_writes():
            for _ in range(16):
                pltpu.make_async_copy(out_hbm.at[0, 0, 0],
                                      tbuf.at[pl.ds(0, 1024)], sem_o).wait()

        def write_out(c):
            s = c % _NSLOT
            cp_b[s].wait()
            cg = base // _CHUNK + c
            f = cg // _CH_PER_F
            j0 = (cg % _CH_PER_F) * 8
            for half in range(2):

                def blk(q, carry, _s=s, _half=half, _f=f, _j0=j0):
                    jl = _half * 4 + q
                    toff = q * 4096

                    @plsc.parallel_loop(0, 128, unroll=8)
                    def _tr(i):
                        v0 = rows[_s][jl * 128 + i, pl.ds(0, _L)]
                        plsc.store_scatter(tbuf, [iota128 + (toff + i)], v0)
                        v1 = rows[_s][jl * 128 + i, pl.ds(_L, _L)]
                        plsc.store_scatter(
                            tbuf, [iota128 + (toff + _L * 128 + i)], v1)

                    for i in range(_DIM // 8):
                        pltpu.async_copy(
                            tbuf.at[pl.ds(toff + 1024 * i, 1024)],
                            out_hbm.at[_f, i, _j0 + jl],
                            sem_o)
                    return carry

                lax.fori_loop(0, 4, blk, 0)
                drain_writes()
            return

        for c in range(_CH_PER_W + 2):
            if 2 <= c:
                write_out(c - 2)
            if c < _CH_PER_W:
                start_a(c)
            if 1 <= c <= _CH_PER_W:
                a_to_b(c - 1)

    return lookup_kernel


def kernel(ids, table):
    tab_t = jnp.transpose(table)
    ids_flat = jnp.transpose(ids).reshape(_TOTAL)
    a_tab = table[: (_NA - 1) * 64 + 1 : 64]
    b1 = _make_stage_kernel()(tab_t)
    out5 = _make_lookup_kernel()(
        ids_flat,
        a_tab,
        b1.reshape(_NB, _DIM),
    )
    out5 = out5.reshape(_FIELDS, _DIM // 8, _BATCH // 128, 8, 128)
    return out5.transpose(2, 4, 0, 1, 3).reshape(_BATCH, _FIELDS, _DIM)
